# Initial kernel scaffold; baseline (speedup 1.0000x reference)
import jax
import jax.numpy as jnp
from jax.experimental import pallas as pl


def kernel(feat_matrix, padding, embed_table):
    raise NotImplementedError("write your pallas kernel here")


# speedup vs baseline: 6.7755x; 6.7755x over previous
import functools

import jax
import jax.numpy as jnp
import numpy as np
from jax import lax
from jax.experimental import pallas as pl
from jax.experimental.pallas import tpu as pltpu
from jax.experimental.pallas import tpu_sc as plsc

B = 1024
L = 50
BL = B * L
NGT = 26
G = 10
D = 32
NF = 100000

NC = 2
NS = 16
NW = NC * NS
LANES = 16

POS_PER_W = BL // NW
CHUNK = 64
NCHUNK = POS_PER_W // CHUNK
GRP = 8
NGRP = CHUNK // GRP
ROWS = CHUNK * G

def _pattern_vec(j):
    i = lax.iota(jnp.int32, LANES)
    r = i + (j * LANES) % G
    q0 = (j * LANES) // G
    c1 = (r >= G).astype(jnp.int32)
    c2 = (r >= 2 * G).astype(jnp.int32)
    q = q0 + c1 + c2
    rr = r - G * (c1 + c2)
    return q * NGT + rr

_mesh = plsc.VectorSubcoreMesh(
    core_axis_name="c", subcore_axis_name="s", num_cores=NC, num_subcores=NS
)


@functools.partial(
    pl.kernel,
    out_type=jax.ShapeDtypeStruct((BL * G, D), jnp.float32),
    mesh=_mesh,
    compiler_params=pltpu.CompilerParams(
        use_tc_tiling_on_sc=False, needs_layout_passes=False),
    scratch_types=[
        pltpu.VMEM((CHUNK * NGT,), jnp.int32),
        pltpu.VMEM((NGRP, GRP * G), jnp.int32),
        pltpu.VMEM((CHUNK,), jnp.float32),
        pltpu.VMEM((ROWS, D), jnp.float32),
        pltpu.SemaphoreType.DMA,
    ],
)
def _feat_embed(feat_hbm, mask_hbm, table_hbm, out_hbm,
                slab_v, idx_v, mask_v, rows_v, sem):
    wid = lax.axis_index("s") * NC + lax.axis_index("c")
    wpos0 = wid * POS_PER_W
    pats = [_pattern_vec(j) for j in range(5)]

    def chunk_body(c, carry):
        pos0 = pl.multiple_of(wpos0 + c * CHUNK, 8)
        pltpu.sync_copy(
            feat_hbm.at[pl.ds(pl.multiple_of(pos0 * NGT, 8), CHUNK * NGT)],
            slab_v)
        pltpu.sync_copy(mask_hbm.at[pl.ds(pos0, CHUNK)], mask_v)

        def grp_body(g8, carry2):
            base = lax.broadcast(g8 * (GRP * NGT), (LANES,))
            for j in range(5):
                vec = plsc.load_gather(slab_v, [pats[j] + base])
                idx_v[g8, pl.ds(j * LANES, LANES)] = vec
            return carry2
        lax.fori_loop(0, NGRP, grp_body, 0, unroll=False)

        copies = [
            pltpu.async_copy(
                table_hbm.at[idx_v.at[g8]],
                rows_v.at[pl.ds(g8 * (GRP * G), GRP * G)],
                sem)
            for g8 in range(NGRP)
        ]
        for cp in copies:
            cp.wait()

        def pos_body(p, carry2):
            m = plsc.load_gather(mask_v, [lax.broadcast(p, (LANES,))])
            for r in range(G):
                row = p * G + r
                for h in (0, LANES):
                    rows_v[row, pl.ds(h, LANES)] = rows_v[row, pl.ds(h, LANES)] * m
            return carry2
        lax.fori_loop(0, CHUNK, pos_body, 0, unroll=False)

        pltpu.sync_copy(
            rows_v, out_hbm.at[pl.ds(pl.multiple_of(pos0 * G, 8), ROWS)])
        return carry

    lax.fori_loop(0, NCHUNK, chunk_body, 0, unroll=False)


def kernel(feat_matrix, padding, embed_table):
    feat_flat = feat_matrix.reshape(-1).astype(jnp.int32)
    maskf = 1.0 - padding.reshape(-1).astype(jnp.float32)
    out = _feat_embed(feat_flat, maskf, embed_table)
    return out.reshape(B, L, G * D)


# speedup vs baseline: 7.6634x; 1.1310x over previous
import functools

import jax
import jax.numpy as jnp
from jax import lax
from jax.experimental import pallas as pl
from jax.experimental.pallas import tpu as pltpu
from jax.experimental.pallas import tpu_sc as plsc

B = 1024
L = 50
BL = B * L
NGT = 26
G = 10
D = 32

NC = 2
NS = 16
NW = NC * NS
LANES = 16

POS_PER_W = BL // NW
CHUNK = 80
NCHUNK = POS_PER_W // CHUNK
GRP = 8
NGRP = CHUNK // GRP
ROWS = CHUNK * G


def _pattern_vec(j):
    i = lax.iota(jnp.int32, LANES)
    r = i + (j * LANES) % G
    q0 = (j * LANES) // G
    c1 = (r >= G).astype(jnp.int32)
    c2 = (r >= 2 * G).astype(jnp.int32)
    q = q0 + c1 + c2
    rr = r - G * (c1 + c2)
    return q * NGT + rr


_mesh = plsc.VectorSubcoreMesh(
    core_axis_name="c", subcore_axis_name="s", num_cores=NC, num_subcores=NS
)


@functools.partial(
    pl.kernel,
    out_type=jax.ShapeDtypeStruct((BL * G, D), jnp.float32),
    mesh=_mesh,
    compiler_params=pltpu.CompilerParams(
        use_tc_tiling_on_sc=False, needs_layout_passes=False),
    scratch_types=[
        pltpu.VMEM((CHUNK * NGT,), jnp.int32),
        pltpu.VMEM((CHUNK * NGT,), jnp.int32),
        pltpu.VMEM((NGRP, GRP * G), jnp.int32),
        pltpu.VMEM((NGRP, GRP * G), jnp.int32),
        pltpu.VMEM((CHUNK,), jnp.float32),
        pltpu.VMEM((CHUNK,), jnp.float32),
        pltpu.VMEM((ROWS, D), jnp.float32),
        pltpu.VMEM((ROWS, D), jnp.float32),
        pltpu.SemaphoreType.DMA,
        pltpu.SemaphoreType.DMA,
        pltpu.SemaphoreType.DMA,
        pltpu.SemaphoreType.DMA,
    ],
)
def _feat_embed(feat_hbm, mask_hbm, table_hbm, out_hbm,
                slab0, slab1, idx0, idx1, mask0, mask1, rows0, rows1,
                semg0, semg1, semo0, semo1):
    slab = (slab0, slab1)
    idx = (idx0, idx1)
    maskv = (mask0, mask1)
    rows = (rows0, rows1)
    semg = (semg0, semg1)
    semo = (semo0, semo1)

    wid = lax.axis_index("s") * NC + lax.axis_index("c")
    wpos0 = wid * POS_PER_W
    pats = [_pattern_vec(j) for j in range(5)]

    def stage_and_fire(c, b):
        pos0 = pl.multiple_of(wpos0 + c * CHUNK, 8)
        pltpu.sync_copy(
            feat_hbm.at[pl.ds(pl.multiple_of(pos0 * NGT, 8), CHUNK * NGT)],
            slab[b])
        pltpu.sync_copy(mask_hbm.at[pl.ds(pos0, CHUNK)], maskv[b])

        def grp_body(g8, carry):
            base = lax.broadcast(g8 * (GRP * NGT), (LANES,))
            for j in range(5):
                vec = plsc.load_gather(slab[b], [pats[j] + base])
                idx[b][g8, pl.ds(j * LANES, LANES)] = vec
            return carry
        lax.fori_loop(0, NGRP, grp_body, 0, unroll=False)

        for g8 in range(NGRP):
            pltpu.async_copy(
                table_hbm.at[idx[b].at[g8]],
                rows[b].at[pl.ds(g8 * (GRP * G), GRP * G)],
                semg[b])

    def wait_gathers(b):
        for g8 in range(NGRP):
            pltpu.make_async_copy(
                table_hbm.at[idx[b].at[g8]],
                rows[b].at[pl.ds(g8 * (GRP * G), GRP * G)],
                semg[b]).wait()

    def out_slice(c):
        row0 = pl.multiple_of((wpos0 + c * CHUNK) * G, 8)
        return out_hbm.at[pl.ds(row0, ROWS)]

    def drain_out(c, b):
        pltpu.make_async_copy(rows[b], out_slice(c), semo[b]).wait()

    stage_and_fire(0, 0)

    def pair_body(cc, carry):
        for b in (0, 1):
            c = cc * 2 + b

            @pl.when(c + 1 < NCHUNK)
            def _fire_next():
                @pl.when(c >= 1)
                def _drain_prev():
                    drain_out(c - 1, 1 - b)
                stage_and_fire(c + 1, 1 - b)

            wait_gathers(b)

            def pos_body(p, carry2):
                m = plsc.load_gather(maskv[b], [lax.broadcast(p, (LANES,))])
                for r in range(G):
                    row = p * G + r
                    for h in (0, LANES):
                        rows[b][row, pl.ds(h, LANES)] = (
                            rows[b][row, pl.ds(h, LANES)] * m)
                return carry2
            lax.fori_loop(0, CHUNK, pos_body, 0, unroll=False)

            pltpu.async_copy(rows[b], out_slice(c), semo[b])
        return carry

    lax.fori_loop(0, NCHUNK // 2, pair_body, 0, unroll=False)
    drain_out(NCHUNK - 2, 0)
    drain_out(NCHUNK - 1, 1)


def kernel(feat_matrix, padding, embed_table):
    feat_flat = feat_matrix.reshape(-1).astype(jnp.int32)
    maskf = 1.0 - padding.reshape(-1).astype(jnp.float32)
    out = _feat_embed(feat_flat, maskf, embed_table)
    return out.reshape(B, L, G * D)


# speedup vs baseline: 7.8983x; 1.0307x over previous
import functools

import jax
import jax.numpy as jnp
from jax import lax
from jax.experimental import pallas as pl
from jax.experimental.pallas import tpu as pltpu
from jax.experimental.pallas import tpu_sc as plsc

B = 1024
L = 50
BL = B * L
G = 10
D = 32

NC = 2
NS = 16
NW = NC * NS
LANES = 16

POS_PER_W = BL // NW
CHUNK = 80
NCHUNK = POS_PER_W // CHUNK
STREAM = 80
NGRP = CHUNK * G // STREAM
ROWS = CHUNK * G

_mesh = plsc.VectorSubcoreMesh(
    core_axis_name="c", subcore_axis_name="s", num_cores=NC, num_subcores=NS
)


@functools.partial(
    pl.kernel,
    out_type=jax.ShapeDtypeStruct((BL * G, D), jnp.float32),
    mesh=_mesh,
    compiler_params=pltpu.CompilerParams(
        use_tc_tiling_on_sc=False, needs_layout_passes=False),
    scratch_types=[
        pltpu.VMEM((NGRP, STREAM), jnp.int32),
        pltpu.VMEM((NGRP, STREAM), jnp.int32),
        pltpu.VMEM((CHUNK,), jnp.float32),
        pltpu.VMEM((CHUNK,), jnp.float32),
        pltpu.VMEM((ROWS, D), jnp.float32),
        pltpu.VMEM((ROWS, D), jnp.float32),
        pltpu.SemaphoreType.DMA,
        pltpu.SemaphoreType.DMA,
        pltpu.SemaphoreType.DMA,
        pltpu.SemaphoreType.DMA,
    ],
)
def _feat_embed(sel_hbm, mask_hbm, table_hbm, out_hbm,
                idx0, idx1, mask0, mask1, rows0, rows1,
                semg0, semg1, semo0, semo1):
    idx = (idx0, idx1)
    maskv = (mask0, mask1)
    rows = (rows0, rows1)
    semg = (semg0, semg1)
    semo = (semo0, semo1)

    wid = lax.axis_index("s") * NC + lax.axis_index("c")
    wpos0 = wid * POS_PER_W

    def stage_and_fire(c, b):
        pos0 = pl.multiple_of(wpos0 + c * CHUNK, 8)
        pltpu.sync_copy(sel_hbm.at[pl.ds(pos0 * G // STREAM, NGRP)], idx[b])
        pltpu.sync_copy(mask_hbm.at[pl.ds(pos0, CHUNK)], maskv[b])
        for g in range(NGRP):
            pltpu.async_copy(
                table_hbm.at[idx[b].at[g]],
                rows[b].at[pl.ds(g * STREAM, STREAM)],
                semg[b])

    def wait_gathers(b):
        for g in range(NGRP):
            pltpu.make_async_copy(
                table_hbm.at[idx[b].at[g]],
                rows[b].at[pl.ds(g * STREAM, STREAM)],
                semg[b]).wait()

    def out_slice(c):
        row0 = pl.multiple_of((wpos0 + c * CHUNK) * G, 8)
        return out_hbm.at[pl.ds(row0, ROWS)]

    def drain_out(c, b):
        pltpu.make_async_copy(rows[b], out_slice(c), semo[b]).wait()

    stage_and_fire(0, 0)

    def pair_body(cc, carry):
        for b in (0, 1):
            c = cc * 2 + b

            @pl.when(c + 1 < NCHUNK)
            def _fire_next():
                @pl.when(c >= 1)
                def _drain_prev():
                    drain_out(c - 1, 1 - b)
                stage_and_fire(c + 1, 1 - b)

            wait_gathers(b)

            def pos_body(p, carry2):
                m = plsc.load_gather(maskv[b], [lax.broadcast(p, (LANES,))])
                for r in range(G):
                    row = p * G + r
                    for h in (0, LANES):
                        rows[b][row, pl.ds(h, LANES)] = (
                            rows[b][row, pl.ds(h, LANES)] * m)
                return carry2
            lax.fori_loop(0, CHUNK, pos_body, 0, unroll=False)

            pltpu.async_copy(rows[b], out_slice(c), semo[b])
        return carry

    lax.fori_loop(0, NCHUNK // 2, pair_body, 0, unroll=False)
    drain_out(NCHUNK - 2, 0)
    drain_out(NCHUNK - 1, 1)


def kernel(feat_matrix, padding, embed_table):
    sel = feat_matrix[:, :, :G].reshape(BL * G // STREAM, STREAM)
    sel = sel.astype(jnp.int32)
    maskf = 1.0 - padding.reshape(-1).astype(jnp.float32)
    out = _feat_embed(sel, maskf, embed_table)
    return out.reshape(B, L, G * D)
